# baseline (speedup 1.0000x reference)
import jax
import jax.numpy as jnp
from jax import lax
from jax.experimental import pallas as pl
from jax.experimental.pallas import tpu as pltpu
from jax.experimental.pallas import tpu_sc as plsc

F = 5
F8 = 8
HID = 64
DIN = 256
DOUT = 256
G = 100
NPG = 100
EPG = 1600
N = G * NPG
E = G * EPG

ROWS = 2000
NW = 32
LANES = 16
GPW = 4
GP = NW * GPW
NPAD = 112
SLAB = GPW * NPG


def _tc1_body(x_ref, w1_ref, b1_ref, w2_ref, b2_ref, fv_ref):
    h = jnp.dot(x_ref[...], w1_ref[...], preferred_element_type=jnp.float32)
    h = h + b1_ref[...]
    h = h * jax.nn.sigmoid(h)
    fv_ref[...] = jnp.dot(h, w2_ref[...], preferred_element_type=jnp.float32) + b2_ref[...]


def _tc1(x, W1, b1, W2p, b2p):
    return pl.pallas_call(
        _tc1_body,
        grid=(N // ROWS,),
        in_specs=[
            pl.BlockSpec((ROWS, DIN), lambda i: (i, 0)),
            pl.BlockSpec((DIN, HID), lambda i: (0, 0)),
            pl.BlockSpec((1, HID), lambda i: (0, 0)),
            pl.BlockSpec((HID, F8), lambda i: (0, 0)),
            pl.BlockSpec((1, F8), lambda i: (0, 0)),
        ],
        out_specs=pl.BlockSpec((ROWS, F8), lambda i: (i, 0)),
        out_shape=jax.ShapeDtypeStruct((N, F8), jnp.float32),
    )(x, W1, b1, W2p, b2p)


def _tc2a_body(x_ref, wox_ref, bo_ref, acc_ref):
    acc_ref[...] = (
        jnp.dot(x_ref[...], wox_ref[...], preferred_element_type=jnp.float32)
        + bo_ref[...]
    )


def _tc2a(x, WoX, bo):
    return pl.pallas_call(
        _tc2a_body,
        grid=(N // ROWS,),
        in_specs=[
            pl.BlockSpec((ROWS, DIN), lambda i: (i, 0)),
            pl.BlockSpec((DIN, DOUT), lambda i: (0, 0)),
            pl.BlockSpec((1, DOUT), lambda i: (0, 0)),
        ],
        out_specs=pl.BlockSpec((ROWS, DOUT), lambda i: (i, 0)),
        out_shape=jax.ShapeDtypeStruct((N, DOUT), jnp.float32),
    )(x, WoX, bo)


def _sc_body(fv_hbm, src_hbm, dst_hbm, d0_hbm, gact_hbm,
             fv_v, src_v, dst_v, d0_v, gact_v, sem, esem, *priv):
    wid = lax.axis_index("s") * 2 + lax.axis_index("c")
    start = wid * GPW
    es = jnp.minimum(start, G - GPW)
    lane = lax.broadcasted_iota(jnp.int32, (LANES,), 0)
    lane_np = lane * NPAD
    inf16 = jnp.full((LANES,), jnp.inf, jnp.float32)
    zero16 = jnp.zeros((LANES,), jnp.float32)
    ninf16 = jnp.full((LANES,), -jnp.inf, jnp.float32)
    fcols = [jnp.full((LANES,), f, jnp.int32) for f in range(F)]

    c1 = pltpu.make_async_copy(
        fv_hbm.at[pl.ds(es * NPG, SLAB)], fv_v.at[pl.ds(0, SLAB)], sem)
    c1.start()

    def fetch_edges(k, buf):
        pltpu.make_async_copy(
            src_hbm.at[pl.ds(pl.multiple_of((es + k) * EPG, 8), EPG)],
            src_v.at[pl.ds(buf * EPG, EPG)], esem).start()
        pltpu.make_async_copy(
            dst_hbm.at[pl.ds(pl.multiple_of((es + k) * EPG, 8), EPG)],
            dst_v.at[pl.ds(buf * EPG, EPG)], esem).start()

    def wait_edges(buf):
        pltpu.make_async_copy(
            src_hbm.at[pl.ds(0, EPG)], src_v.at[pl.ds(buf * EPG, EPG)], esem).wait()
        pltpu.make_async_copy(
            dst_hbm.at[pl.ds(0, EPG)], dst_v.at[pl.ds(buf * EPG, EPG)], esem).wait()

    fetch_edges(0, 0)

    def z_body(j, cc):
        idx = j * 16 + lane
        plsc.store_scatter(d0_v, [idx >> 3, idx & 7], zero16)
        return cc
    lax.fori_loop(0, (SLAB + LANES) * F8 // 16, z_body, 0)

    def init_body(j, cc):
        for p in priv:
            p[pl.ds(j * 16, 16)] = inf16
        return cc
    lax.fori_loop(0, LANES * NPAD // 16, init_body, 0)

    c1.wait()

    def do_graph(k, c):
        gbase = (es + k) * NPG
        buf = lax.rem(k, 2)
        ebase = buf * EPG
        rbase = k * NPG

        wait_edges(buf)

        @pl.when(k < GPW - 1)
        def _():
            fetch_edges(k + 1, 1 - buf)

        def pa(i, cc):
            for h in range(2):
                eo = ebase + h * (EPG // 2) + i * 16
                s = src_v[pl.ds(eo, 16)]
                t = dst_v[pl.ds(eo, 16)]
                sl = s - gbase
                tl = t - gbase
                srow = rbase + sl
                trow = rbase + tl
                ips = lane_np + sl
                ipt = lane_np + tl
                for f in range(F):
                    a = plsc.load_gather(fv_v, [srow, fcols[f]])
                    b = plsc.load_gather(fv_v, [trow, fcols[f]])
                    fe = jnp.maximum(a, b)
                    pr = priv[h * F + f]
                    cs = plsc.load_gather(pr, [ips])
                    plsc.store_scatter(pr, [ips], jnp.minimum(cs, fe))
                    ct = plsc.load_gather(pr, [ipt])
                    plsc.store_scatter(pr, [ipt], jnp.minimum(ct, fe))
            return cc
        lax.fori_loop(0, EPG // 32, pa, 0)

        def rd(j, cc):
            nidx = j * 16 + lane
            rows = rbase + nidx
            for f in range(F):
                m = inf16
                for h in range(2):
                    pr = priv[h * F + f]
                    for l in range(LANES):
                        m = jnp.minimum(m, pr[pl.ds(l * NPAD + j * 16, 16)])
                    for l in range(LANES):
                        pr[pl.ds(l * NPAD + j * 16, 16)] = inf16
                fv16 = plsc.load_gather(fv_v, [rows, fcols[f]])
                m = jnp.where(m == inf16, fv16, m)
                plsc.store_scatter(d0_v, [rows, fcols[f]], m)
            return cc
        lax.fori_loop(0, NPAD // 16, rd, 0)

        def pb(i, carry):
            births, cnts, gmaxs = carry
            nb, nc, ng = list(births), list(cnts), list(gmaxs)
            for h in range(2):
                eo = ebase + h * (EPG // 2) + i * 16
                s = src_v[pl.ds(eo, 16)]
                t = dst_v[pl.ds(eo, 16)]
                srow = rbase + s - gbase
                trow = rbase + t - gbase
                for f in range(F):
                    a = plsc.load_gather(fv_v, [srow, fcols[f]])
                    b = plsc.load_gather(fv_v, [trow, fcols[f]])
                    fe = jnp.maximum(a, b)
                    dsv = plsc.load_gather(d0_v, [srow, fcols[f]])
                    dtv = plsc.load_gather(d0_v, [trow, fcols[f]])
                    cyc = fe > jnp.maximum(dsv, dtv)
                    nb[f] = nb[f] + jnp.where(cyc, fe, 0.0)
                    nc[f] = nc[f] + jnp.where(cyc, 1.0, 0.0)
                    ng[f] = jnp.maximum(ng[f], fe)
            return (tuple(nb), tuple(nc), tuple(ng))

        carry0 = ((zero16,) * F, (zero16,) * F, (ninf16,) * F)
        births, cnts, gmaxs = lax.fori_loop(0, EPG // 32, pb, carry0)

        v = zero16
        for f in range(F):
            bsum = jnp.sum(births[f])
            dsum = jnp.max(gmaxs[f]) * jnp.sum(cnts[f])
            v = jnp.where(lane == 2 * f, bsum, v)
            v = jnp.where(lane == 2 * f + 1, dsum, v)
        gact_v[pl.ds(k * 16, 16)] = v
        return c

    lax.fori_loop(0, GPW, do_graph, 0)

    o1 = pltpu.make_async_copy(
        d0_v.at[pl.ds(0, SLAB)], d0_hbm.at[pl.ds(start * NPG, SLAB)], sem)
    o2 = pltpu.make_async_copy(
        gact_v, gact_hbm.at[pl.ds(pl.multiple_of(start * 16, 8), GPW * 16)], sem)
    o1.start(); o2.start()
    o1.wait(); o2.wait()


def _sc_edges(fv8, src, dst):
    fn = pl.kernel(
        _sc_body,
        out_type=[
            jax.ShapeDtypeStruct((GP * NPG, F8), jnp.float32),
            jax.ShapeDtypeStruct((GP * 16,), jnp.float32),
        ],
        mesh=plsc.VectorSubcoreMesh(core_axis_name="c", subcore_axis_name="s"),
        compiler_params=pltpu.CompilerParams(needs_layout_passes=False),
        scratch_types=[
            pltpu.VMEM((SLAB + LANES, F8), jnp.float32),
            pltpu.VMEM((2 * EPG,), jnp.int32),
            pltpu.VMEM((2 * EPG,), jnp.int32),
            pltpu.VMEM((SLAB + LANES, F8), jnp.float32),
            pltpu.VMEM((GPW * 16,), jnp.float32),
            pltpu.SemaphoreType.DMA,
            pltpu.SemaphoreType.DMA,
        ] + [pltpu.VMEM((LANES * NPAD,), jnp.float32) for _ in range(2 * F)],
    )
    return fn(fv8, src, dst)


def _tc2b_body(acc_ref, fv_ref, d0_ref, wga_ref, wgb_ref,
               mua_ref, nia_ref, mub_ref, nib_ref, out_ref):
    ga = jnp.exp(nia_ref[...] * (fv_ref[...] - mua_ref[...]) ** 2)
    gb = jnp.exp(nib_ref[...] * (d0_ref[...] - mub_ref[...]) ** 2)
    a = acc_ref[...]
    a = a + jnp.dot(ga, wga_ref[...], preferred_element_type=jnp.float32)
    a = a + jnp.dot(gb, wgb_ref[...], preferred_element_type=jnp.float32)
    out_ref[...] = a * jax.nn.sigmoid(a)


def _tc2b(acc, fv8, d08, WGA, WGB, muA, niA, muB, niB):
    return pl.pallas_call(
        _tc2b_body,
        grid=(N // ROWS,),
        in_specs=[
            pl.BlockSpec((ROWS, DOUT), lambda i: (i, 0)),
            pl.BlockSpec((ROWS, F8), lambda i: (i, 0)),
            pl.BlockSpec((ROWS, F8), lambda i: (i, 0)),
            pl.BlockSpec((F8, DOUT), lambda i: (0, 0)),
            pl.BlockSpec((F8, DOUT), lambda i: (0, 0)),
            pl.BlockSpec((1, F8), lambda i: (0, 0)),
            pl.BlockSpec((1, F8), lambda i: (0, 0)),
            pl.BlockSpec((1, F8), lambda i: (0, 0)),
            pl.BlockSpec((1, F8), lambda i: (0, 0)),
        ],
        out_specs=pl.BlockSpec((ROWS, DOUT), lambda i: (i, 0)),
        out_shape=jax.ShapeDtypeStruct((N, DOUT), jnp.float32),
    )(acc, fv8, d08, WGA, WGB, muA, niA, muB, niB)


def kernel(x, edge_index, x_slices, edge_slices, W1, b1, W2, b2, mu, sigma, Wo, bo):
    f32 = jnp.float32
    W2p = jnp.zeros((HID, F8), f32).at[:, :F].set(W2)
    b2p = jnp.zeros((1, F8), f32).at[0, :F].set(b2)
    fv8 = _tc1(x, W1, b1.reshape(1, HID), W2p, b2p)
    acc = _tc2a(x, Wo[:DIN], bo.reshape(1, DOUT))

    d08p, gact_flat = _sc_edges(fv8, edge_index[0], edge_index[1])
    d08 = d08p[:N]

    muA = jnp.zeros((1, F8), f32).at[0, :F].set(mu[0::2])
    muB = jnp.zeros((1, F8), f32).at[0, :F].set(mu[1::2])
    niA = jnp.zeros((1, F8), f32).at[0, :F].set(-0.5 / (sigma[0::2] ** 2))
    niB = jnp.zeros((1, F8), f32).at[0, :F].set(-0.5 / (sigma[1::2] ** 2))
    WGA = jnp.zeros((F8, DOUT), f32).at[:F].set(Wo[DIN::2])
    WGB = jnp.zeros((F8, DOUT), f32).at[:F].set(Wo[DIN + 1::2])

    out = _tc2b(acc, fv8, d08, WGA, WGB, muA, niA, muB, niB)
    gact = gact_flat.reshape(GP, 16)[:G, : 2 * F]
    return out, gact


# speedup vs baseline: 1.4821x; 1.4821x over previous
import jax
import jax.numpy as jnp
from jax import lax
from jax.experimental import pallas as pl
from jax.experimental.pallas import tpu as pltpu
from jax.experimental.pallas import tpu_sc as plsc

F = 5
F8 = 8
HID = 64
DIN = 256
DOUT = 256
G = 100
NPG = 100
EPG = 1600
N = G * NPG
E = G * EPG

ROWS = 2000
NW = 32
LANES = 16
GPW = 4
GP = NW * GPW
NPAD = 112
SLAB = GPW * NPG


def _tc1_body(x_ref, w1_ref, b1_ref, w2_ref, b2_ref, fv_ref):
    h = jnp.dot(x_ref[...], w1_ref[...], preferred_element_type=jnp.float32)
    h = h + b1_ref[...]
    h = h * jax.nn.sigmoid(h)
    fv_ref[...] = jnp.dot(h, w2_ref[...], preferred_element_type=jnp.float32) + b2_ref[...]


def _tc1(x, W1, b1, W2p, b2p):
    return pl.pallas_call(
        _tc1_body,
        grid=(N // ROWS,),
        in_specs=[
            pl.BlockSpec((ROWS, DIN), lambda i: (i, 0)),
            pl.BlockSpec((DIN, HID), lambda i: (0, 0)),
            pl.BlockSpec((1, HID), lambda i: (0, 0)),
            pl.BlockSpec((HID, F8), lambda i: (0, 0)),
            pl.BlockSpec((1, F8), lambda i: (0, 0)),
        ],
        out_specs=pl.BlockSpec((ROWS, F8), lambda i: (i, 0)),
        out_shape=jax.ShapeDtypeStruct((N, F8), jnp.float32),
    )(x, W1, b1, W2p, b2p)


def _tc2a_body(x_ref, wox_ref, bo_ref, acc_ref):
    acc_ref[...] = (
        jnp.dot(x_ref[...], wox_ref[...], preferred_element_type=jnp.float32)
        + bo_ref[...]
    )


def _tc2a(x, WoX, bo):
    return pl.pallas_call(
        _tc2a_body,
        grid=(N // ROWS,),
        in_specs=[
            pl.BlockSpec((ROWS, DIN), lambda i: (i, 0)),
            pl.BlockSpec((DIN, DOUT), lambda i: (0, 0)),
            pl.BlockSpec((1, DOUT), lambda i: (0, 0)),
        ],
        out_specs=pl.BlockSpec((ROWS, DOUT), lambda i: (i, 0)),
        out_shape=jax.ShapeDtypeStruct((N, DOUT), jnp.float32),
    )(x, WoX, bo)


def _sc_body(fv_hbm, src_hbm, dst_hbm, d0_hbm, gact_hbm,
             fv_v, src_v, dst_v, d0_v, gact_v, sem, esem, *priv):
    wid = lax.axis_index("s") * 2 + lax.axis_index("c")
    start = wid * GPW
    es = jnp.minimum(start, G - GPW)
    lane = lax.broadcasted_iota(jnp.int32, (LANES,), 0)
    lane_np = lane * NPAD
    inf16 = jnp.full((LANES,), jnp.inf, jnp.float32)
    zero16 = jnp.zeros((LANES,), jnp.float32)
    ninf16 = jnp.full((LANES,), -jnp.inf, jnp.float32)
    fcols = [jnp.full((LANES,), f, jnp.int32) for f in range(F)]

    c1 = pltpu.make_async_copy(
        fv_hbm.at[pl.ds(es * NPG, SLAB)], fv_v.at[pl.ds(0, SLAB)], sem)
    c1.start()

    def fetch_edges(k, buf):
        pltpu.make_async_copy(
            src_hbm.at[pl.ds(pl.multiple_of((es + k) * EPG, 8), EPG)],
            src_v.at[pl.ds(buf * EPG, EPG)], esem).start()
        pltpu.make_async_copy(
            dst_hbm.at[pl.ds(pl.multiple_of((es + k) * EPG, 8), EPG)],
            dst_v.at[pl.ds(buf * EPG, EPG)], esem).start()

    def wait_edges(buf):
        pltpu.make_async_copy(
            src_hbm.at[pl.ds(0, EPG)], src_v.at[pl.ds(buf * EPG, EPG)], esem).wait()
        pltpu.make_async_copy(
            dst_hbm.at[pl.ds(0, EPG)], dst_v.at[pl.ds(buf * EPG, EPG)], esem).wait()

    fetch_edges(0, 0)

    def z_body(j, cc):
        idx = j * 16 + lane
        plsc.store_scatter(d0_v, [idx >> 3, idx & 7], zero16)
        return cc
    lax.fori_loop(0, (SLAB + LANES) * F8 // 16, z_body, 0)

    def init_body(j, cc):
        for p in priv:
            p[pl.ds(j * 16, 16)] = inf16
        return cc
    lax.fori_loop(0, LANES * NPAD // 16, init_body, 0)

    c1.wait()

    def do_graph(k, c):
        gbase = (es + k) * NPG
        buf = lax.rem(k, 2)
        ebase = buf * EPG
        rbase = k * NPG

        wait_edges(buf)

        @pl.when(k < GPW - 1)
        def _():
            fetch_edges(k + 1, 1 - buf)

        def pa(i, cc):
            for h in range(2):
                eo = ebase + h * (EPG // 2) + i * 16
                s = src_v[pl.ds(eo, 16)]
                t = dst_v[pl.ds(eo, 16)]
                sl = s - gbase
                tl = t - gbase
                srow = rbase + sl
                trow = rbase + tl
                ips = lane_np + sl
                ipt = lane_np + tl
                for f in range(F):
                    a = plsc.load_gather(fv_v, [srow, fcols[f]])
                    b = plsc.load_gather(fv_v, [trow, fcols[f]])
                    fe = jnp.maximum(a, b)
                    pr = priv[h * F + f]
                    cs = plsc.load_gather(pr, [ips])
                    plsc.store_scatter(pr, [ips], jnp.minimum(cs, fe))
                    ct = plsc.load_gather(pr, [ipt])
                    plsc.store_scatter(pr, [ipt], jnp.minimum(ct, fe))
            return cc
        lax.fori_loop(0, EPG // 32, pa, 0)

        def rd(j, cc):
            nidx = j * 16 + lane
            rows = rbase + nidx
            for f in range(F):
                m = inf16
                for h in range(2):
                    pr = priv[h * F + f]
                    for l in range(LANES):
                        m = jnp.minimum(m, pr[pl.ds(l * NPAD + j * 16, 16)])
                    for l in range(LANES):
                        pr[pl.ds(l * NPAD + j * 16, 16)] = inf16
                fv16 = plsc.load_gather(fv_v, [rows, fcols[f]])
                m = jnp.where(m == inf16, fv16, m)
                plsc.store_scatter(d0_v, [rows, fcols[f]], m)
            return cc
        lax.fori_loop(0, NPAD // 16, rd, 0)

        def pb(i, carry):
            births, cnts, gmaxs = carry
            nb, nc, ng = list(births), list(cnts), list(gmaxs)
            for h in range(2):
                eo = ebase + h * (EPG // 2) + i * 16
                s = src_v[pl.ds(eo, 16)]
                t = dst_v[pl.ds(eo, 16)]
                srow = rbase + s - gbase
                trow = rbase + t - gbase
                for f in range(F):
                    a = plsc.load_gather(fv_v, [srow, fcols[f]])
                    b = plsc.load_gather(fv_v, [trow, fcols[f]])
                    fe = jnp.maximum(a, b)
                    dsv = plsc.load_gather(d0_v, [srow, fcols[f]])
                    dtv = plsc.load_gather(d0_v, [trow, fcols[f]])
                    cyc = fe > jnp.maximum(dsv, dtv)
                    nb[f] = nb[f] + jnp.where(cyc, fe, 0.0)
                    nc[f] = nc[f] + jnp.where(cyc, 1.0, 0.0)
                    ng[f] = jnp.maximum(ng[f], fe)
            return (tuple(nb), tuple(nc), tuple(ng))

        carry0 = ((zero16,) * F, (zero16,) * F, (ninf16,) * F)
        births, cnts, gmaxs = lax.fori_loop(0, EPG // 32, pb, carry0)

        v = zero16
        for f in range(F):
            bsum = jnp.sum(births[f])
            dsum = jnp.max(gmaxs[f]) * jnp.sum(cnts[f])
            v = jnp.where(lane == 2 * f, bsum, v)
            v = jnp.where(lane == 2 * f + 1, dsum, v)
        gact_v[pl.ds(k * 16, 16)] = v
        return c

    lax.fori_loop(0, GPW, do_graph, 0)

    o1 = pltpu.make_async_copy(
        d0_v.at[pl.ds(0, SLAB)], d0_hbm.at[pl.ds(start * NPG, SLAB)], sem)
    o2 = pltpu.make_async_copy(
        gact_v, gact_hbm.at[pl.ds(pl.multiple_of(start * 16, 8), GPW * 16)], sem)
    o1.start(); o2.start()
    o1.wait(); o2.wait()


def _sc_edges(fv8, src, dst):
    fn = pl.kernel(
        _sc_body,
        out_type=[
            jax.ShapeDtypeStruct((GP * NPG, F8), jnp.float32),
            jax.ShapeDtypeStruct((GP * 16,), jnp.float32),
        ],
        mesh=plsc.VectorSubcoreMesh(core_axis_name="c", subcore_axis_name="s"),
        compiler_params=pltpu.CompilerParams(
            needs_layout_passes=False, use_tc_tiling_on_sc=False),
        scratch_types=[
            pltpu.VMEM((SLAB + LANES, F8), jnp.float32),
            pltpu.VMEM((2 * EPG,), jnp.int32),
            pltpu.VMEM((2 * EPG,), jnp.int32),
            pltpu.VMEM((SLAB + LANES, F8), jnp.float32),
            pltpu.VMEM((GPW * 16,), jnp.float32),
            pltpu.SemaphoreType.DMA,
            pltpu.SemaphoreType.DMA,
        ] + [pltpu.VMEM((LANES * NPAD,), jnp.float32) for _ in range(2 * F)],
    )
    return fn(fv8, src, dst)


def _tc2b_body(acc_ref, fv_ref, d0_ref, wga_ref, wgb_ref,
               mua_ref, nia_ref, mub_ref, nib_ref, out_ref):
    ga = jnp.exp(nia_ref[...] * (fv_ref[...] - mua_ref[...]) ** 2)
    gb = jnp.exp(nib_ref[...] * (d0_ref[...] - mub_ref[...]) ** 2)
    a = acc_ref[...]
    a = a + jnp.dot(ga, wga_ref[...], preferred_element_type=jnp.float32)
    a = a + jnp.dot(gb, wgb_ref[...], preferred_element_type=jnp.float32)
    out_ref[...] = a * jax.nn.sigmoid(a)


def _tc2b(acc, fv8, d08, WGA, WGB, muA, niA, muB, niB):
    return pl.pallas_call(
        _tc2b_body,
        grid=(N // ROWS,),
        in_specs=[
            pl.BlockSpec((ROWS, DOUT), lambda i: (i, 0)),
            pl.BlockSpec((ROWS, F8), lambda i: (i, 0)),
            pl.BlockSpec((ROWS, F8), lambda i: (i, 0)),
            pl.BlockSpec((F8, DOUT), lambda i: (0, 0)),
            pl.BlockSpec((F8, DOUT), lambda i: (0, 0)),
            pl.BlockSpec((1, F8), lambda i: (0, 0)),
            pl.BlockSpec((1, F8), lambda i: (0, 0)),
            pl.BlockSpec((1, F8), lambda i: (0, 0)),
            pl.BlockSpec((1, F8), lambda i: (0, 0)),
        ],
        out_specs=pl.BlockSpec((ROWS, DOUT), lambda i: (i, 0)),
        out_shape=jax.ShapeDtypeStruct((N, DOUT), jnp.float32),
    )(acc, fv8, d08, WGA, WGB, muA, niA, muB, niB)


def kernel(x, edge_index, x_slices, edge_slices, W1, b1, W2, b2, mu, sigma, Wo, bo):
    f32 = jnp.float32
    W2p = jnp.zeros((HID, F8), f32).at[:, :F].set(W2)
    b2p = jnp.zeros((1, F8), f32).at[0, :F].set(b2)
    fv8 = _tc1(x, W1, b1.reshape(1, HID), W2p, b2p)
    acc = _tc2a(x, Wo[:DIN], bo.reshape(1, DOUT))

    d08p, gact_flat = _sc_edges(fv8, edge_index[0], edge_index[1])
    d08 = d08p[:N]

    muA = jnp.zeros((1, F8), f32).at[0, :F].set(mu[0::2])
    muB = jnp.zeros((1, F8), f32).at[0, :F].set(mu[1::2])
    niA = jnp.zeros((1, F8), f32).at[0, :F].set(-0.5 / (sigma[0::2] ** 2))
    niB = jnp.zeros((1, F8), f32).at[0, :F].set(-0.5 / (sigma[1::2] ** 2))
    WGA = jnp.zeros((F8, DOUT), f32).at[:F].set(Wo[DIN::2])
    WGB = jnp.zeros((F8, DOUT), f32).at[:F].set(Wo[DIN + 1::2])

    out = _tc2b(acc, fv8, d08, WGA, WGB, muA, niA, muB, niB)
    gact = gact_flat.reshape(GP, 16)[:G, : 2 * F]
    return out, gact


# speedup vs baseline: 1.8082x; 1.2201x over previous
import jax
import jax.numpy as jnp
from jax import lax
from jax.experimental import pallas as pl
from jax.experimental.pallas import tpu as pltpu
from jax.experimental.pallas import tpu_sc as plsc

F = 5
F8 = 8
HID = 64
DIN = 256
DOUT = 256
G = 100
NPG = 100
EPG = 1600
N = G * NPG
E = G * EPG

ROWS = 2000
NW = 32
LANES = 16
GPW = 4
GP = NW * GPW
NPAD = 112
SLAB = GPW * NPG


def _tc1_body(x_ref, w1_ref, b1_ref, w2_ref, b2_ref, fv_ref):
    h = jnp.dot(x_ref[...], w1_ref[...], preferred_element_type=jnp.float32)
    h = h + b1_ref[...]
    h = h * jax.nn.sigmoid(h)
    fv_ref[...] = jnp.dot(h, w2_ref[...], preferred_element_type=jnp.float32) + b2_ref[...]


def _tc1(x, W1, b1, W2p, b2p):
    return pl.pallas_call(
        _tc1_body,
        grid=(N // ROWS,),
        in_specs=[
            pl.BlockSpec((ROWS, DIN), lambda i: (i, 0)),
            pl.BlockSpec((DIN, HID), lambda i: (0, 0)),
            pl.BlockSpec((1, HID), lambda i: (0, 0)),
            pl.BlockSpec((HID, F8), lambda i: (0, 0)),
            pl.BlockSpec((1, F8), lambda i: (0, 0)),
        ],
        out_specs=pl.BlockSpec((ROWS, F8), lambda i: (i, 0)),
        out_shape=jax.ShapeDtypeStruct((N, F8), jnp.float32),
    )(x, W1, b1, W2p, b2p)


def _tc2a_body(x_ref, wox_ref, bo_ref, acc_ref):
    acc_ref[...] = (
        jnp.dot(x_ref[...], wox_ref[...], preferred_element_type=jnp.float32)
        + bo_ref[...]
    )


def _tc2a(x, WoX, bo):
    return pl.pallas_call(
        _tc2a_body,
        grid=(N // ROWS,),
        in_specs=[
            pl.BlockSpec((ROWS, DIN), lambda i: (i, 0)),
            pl.BlockSpec((DIN, DOUT), lambda i: (0, 0)),
            pl.BlockSpec((1, DOUT), lambda i: (0, 0)),
        ],
        out_specs=pl.BlockSpec((ROWS, DOUT), lambda i: (i, 0)),
        out_shape=jax.ShapeDtypeStruct((N, DOUT), jnp.float32),
    )(x, WoX, bo)


def _sc_body(fv_hbm, src_hbm, dst_hbm, d0_hbm, gact_hbm,
             fv_v, src_v, dst_v, fe_v, d0_v, gact_v, sem, esem, *priv):
    wid = lax.axis_index("s") * 2 + lax.axis_index("c")
    start = wid * GPW
    es = jnp.minimum(start, G - GPW)
    lane = lax.broadcasted_iota(jnp.int32, (LANES,), 0)
    lane_np = lane * NPAD
    inf16 = jnp.full((LANES,), jnp.inf, jnp.float32)
    zero16 = jnp.zeros((LANES,), jnp.float32)
    ninf16 = jnp.full((LANES,), -jnp.inf, jnp.float32)
    fcols = [jnp.full((LANES,), f, jnp.int32) for f in range(F)]

    c1 = pltpu.make_async_copy(
        fv_hbm.at[pl.ds(es * NPG, SLAB)], fv_v.at[pl.ds(0, SLAB)], sem)
    c1.start()

    def fetch_edges(k, buf):
        pltpu.make_async_copy(
            src_hbm.at[pl.ds(pl.multiple_of((es + k) * EPG, 8), EPG)],
            src_v.at[pl.ds(buf * EPG, EPG)], esem).start()
        pltpu.make_async_copy(
            dst_hbm.at[pl.ds(pl.multiple_of((es + k) * EPG, 8), EPG)],
            dst_v.at[pl.ds(buf * EPG, EPG)], esem).start()

    def wait_edges(buf):
        pltpu.make_async_copy(
            src_hbm.at[pl.ds(0, EPG)], src_v.at[pl.ds(buf * EPG, EPG)], esem).wait()
        pltpu.make_async_copy(
            dst_hbm.at[pl.ds(0, EPG)], dst_v.at[pl.ds(buf * EPG, EPG)], esem).wait()

    fetch_edges(0, 0)

    def z_body(j, cc):
        idx = j * 16 + lane
        plsc.store_scatter(d0_v, [idx >> 3, idx & 7], zero16)
        return cc
    lax.fori_loop(0, (SLAB + LANES) * F8 // 16, z_body, 0)

    def init_body(j, cc):
        for p in priv:
            p[pl.ds(j * 16, 16)] = inf16
        return cc
    lax.fori_loop(0, LANES * NPAD // 16, init_body, 0)

    c1.wait()

    def do_graph(k, c):
        gbase = (es + k) * NPG
        buf = lax.rem(k, 2)
        ebase = buf * EPG
        rbase = k * NPG

        wait_edges(buf)

        @pl.when(k < GPW - 1)
        def _():
            fetch_edges(k + 1, 1 - buf)

        def pa(i, cc):
            eo = ebase + i * 16
            s = src_v[pl.ds(eo, 16)]
            t = dst_v[pl.ds(eo, 16)]
            sl = s - gbase
            tl = t - gbase
            srow = rbase + sl
            trow = rbase + tl
            ips = lane_np + sl
            ipt = lane_np + tl
            fes, css, cts = [], [], []
            for f in range(F):
                a = plsc.load_gather(fv_v, [srow, fcols[f]])
                b = plsc.load_gather(fv_v, [trow, fcols[f]])
                fes.append(jnp.maximum(a, b))
            for f in range(F):
                css.append(plsc.load_gather(priv[f], [ips]))
                cts.append(plsc.load_gather(priv[F + f], [ipt]))
            for f in range(F):
                fe_v[pl.ds(f * EPG + i * 16, 16)] = fes[f]
            for f in range(F):
                plsc.store_scatter(priv[f], [ips], jnp.minimum(css[f], fes[f]))
                plsc.store_scatter(priv[F + f], [ipt], jnp.minimum(cts[f], fes[f]))
            return cc
        lax.fori_loop(0, EPG // 16, pa, 0)

        @plsc.parallel_loop(0, NPAD // 16)
        def rd(j):
            nidx = j * 16 + lane
            rows = rbase + nidx
            for f in range(F):
                vals = []
                for h in range(2):
                    pr = priv[h * F + f]
                    for l in range(LANES):
                        vals.append(pr[pl.ds(l * NPAD + j * 16, 16)])
                while len(vals) > 1:
                    vals = [jnp.minimum(vals[p], vals[p + 1])
                            for p in range(0, len(vals) - 1, 2)] + (
                                [vals[-1]] if len(vals) % 2 else [])
                m = vals[0]
                for h in range(2):
                    pr = priv[h * F + f]
                    for l in range(LANES):
                        pr[pl.ds(l * NPAD + j * 16, 16)] = inf16
                fv16 = plsc.load_gather(fv_v, [rows, fcols[f]])
                m = jnp.where(m == inf16, fv16, m)
                plsc.store_scatter(d0_v, [rows, fcols[f]], m)

        carry0 = ((zero16,) * F, (zero16,) * F, (ninf16,) * F)

        @plsc.parallel_loop(0, EPG // 16, carry=carry0)
        def pbout(i, carry):
            births, cnts, gmaxs = carry
            nb, nc, ng = list(births), list(cnts), list(gmaxs)
            eo = ebase + i * 16
            s = src_v[pl.ds(eo, 16)]
            t = dst_v[pl.ds(eo, 16)]
            srow = rbase + s - gbase
            trow = rbase + t - gbase
            for f in range(F):
                fe = fe_v[pl.ds(f * EPG + i * 16, 16)]
                dsv = plsc.load_gather(d0_v, [srow, fcols[f]])
                dtv = plsc.load_gather(d0_v, [trow, fcols[f]])
                cyc = fe > jnp.maximum(dsv, dtv)
                nb[f] = nb[f] + jnp.where(cyc, fe, 0.0)
                nc[f] = nc[f] + jnp.where(cyc, 1.0, 0.0)
                ng[f] = jnp.maximum(ng[f], fe)
            return (tuple(nb), tuple(nc), tuple(ng))

        births, cnts, gmaxs = pbout

        v = zero16
        for f in range(F):
            bsum = jnp.sum(births[f])
            dsum = jnp.max(gmaxs[f]) * jnp.sum(cnts[f])
            v = jnp.where(lane == 2 * f, bsum, v)
            v = jnp.where(lane == 2 * f + 1, dsum, v)
        gact_v[pl.ds(k * 16, 16)] = v
        return c

    lax.fori_loop(0, GPW, do_graph, 0)

    o1 = pltpu.make_async_copy(
        d0_v.at[pl.ds(0, SLAB)], d0_hbm.at[pl.ds(start * NPG, SLAB)], sem)
    o2 = pltpu.make_async_copy(
        gact_v, gact_hbm.at[pl.ds(pl.multiple_of(start * 16, 8), GPW * 16)], sem)
    o1.start(); o2.start()
    o1.wait(); o2.wait()


def _sc_edges(fv8, src, dst):
    fn = pl.kernel(
        _sc_body,
        out_type=[
            jax.ShapeDtypeStruct((GP * NPG, F8), jnp.float32),
            jax.ShapeDtypeStruct((GP * 16,), jnp.float32),
        ],
        mesh=plsc.VectorSubcoreMesh(core_axis_name="c", subcore_axis_name="s"),
        compiler_params=pltpu.CompilerParams(
            needs_layout_passes=False, use_tc_tiling_on_sc=False),
        scratch_types=[
            pltpu.VMEM((SLAB + LANES, F8), jnp.float32),
            pltpu.VMEM((2 * EPG,), jnp.int32),
            pltpu.VMEM((2 * EPG,), jnp.int32),
            pltpu.VMEM((F * EPG,), jnp.float32),
            pltpu.VMEM((SLAB + LANES, F8), jnp.float32),
            pltpu.VMEM((GPW * 16,), jnp.float32),
            pltpu.SemaphoreType.DMA,
            pltpu.SemaphoreType.DMA,
        ] + [pltpu.VMEM((LANES * NPAD,), jnp.float32) for _ in range(2 * F)],
    )
    return fn(fv8, src, dst)


def _tc2b_body(acc_ref, fv_ref, d0_ref, wga_ref, wgb_ref,
               mua_ref, nia_ref, mub_ref, nib_ref, out_ref):
    ga = jnp.exp(nia_ref[...] * (fv_ref[...] - mua_ref[...]) ** 2)
    gb = jnp.exp(nib_ref[...] * (d0_ref[...] - mub_ref[...]) ** 2)
    a = acc_ref[...]
    a = a + jnp.dot(ga, wga_ref[...], preferred_element_type=jnp.float32)
    a = a + jnp.dot(gb, wgb_ref[...], preferred_element_type=jnp.float32)
    out_ref[...] = a * jax.nn.sigmoid(a)


def _tc2b(acc, fv8, d08, WGA, WGB, muA, niA, muB, niB):
    return pl.pallas_call(
        _tc2b_body,
        grid=(N // ROWS,),
        in_specs=[
            pl.BlockSpec((ROWS, DOUT), lambda i: (i, 0)),
            pl.BlockSpec((ROWS, F8), lambda i: (i, 0)),
            pl.BlockSpec((ROWS, F8), lambda i: (i, 0)),
            pl.BlockSpec((F8, DOUT), lambda i: (0, 0)),
            pl.BlockSpec((F8, DOUT), lambda i: (0, 0)),
            pl.BlockSpec((1, F8), lambda i: (0, 0)),
            pl.BlockSpec((1, F8), lambda i: (0, 0)),
            pl.BlockSpec((1, F8), lambda i: (0, 0)),
            pl.BlockSpec((1, F8), lambda i: (0, 0)),
        ],
        out_specs=pl.BlockSpec((ROWS, DOUT), lambda i: (i, 0)),
        out_shape=jax.ShapeDtypeStruct((N, DOUT), jnp.float32),
    )(acc, fv8, d08, WGA, WGB, muA, niA, muB, niB)


def kernel(x, edge_index, x_slices, edge_slices, W1, b1, W2, b2, mu, sigma, Wo, bo):
    f32 = jnp.float32
    W2p = jnp.zeros((HID, F8), f32).at[:, :F].set(W2)
    b2p = jnp.zeros((1, F8), f32).at[0, :F].set(b2)
    fv8 = _tc1(x, W1, b1.reshape(1, HID), W2p, b2p)
    acc = _tc2a(x, Wo[:DIN], bo.reshape(1, DOUT))

    d08p, gact_flat = _sc_edges(fv8, edge_index[0], edge_index[1])
    d08 = d08p[:N]

    muA = jnp.zeros((1, F8), f32).at[0, :F].set(mu[0::2])
    muB = jnp.zeros((1, F8), f32).at[0, :F].set(mu[1::2])
    niA = jnp.zeros((1, F8), f32).at[0, :F].set(-0.5 / (sigma[0::2] ** 2))
    niB = jnp.zeros((1, F8), f32).at[0, :F].set(-0.5 / (sigma[1::2] ** 2))
    WGA = jnp.zeros((F8, DOUT), f32).at[:F].set(Wo[DIN::2])
    WGB = jnp.zeros((F8, DOUT), f32).at[:F].set(Wo[DIN + 1::2])

    out = _tc2b(acc, fv8, d08, WGA, WGB, muA, niA, muB, niB)
    gact = gact_flat.reshape(GP, 16)[:G, : 2 * F]
    return out, gact


# speedup vs baseline: 2.5609x; 1.4163x over previous
import jax
import jax.numpy as jnp
from jax import lax
from jax.experimental import pallas as pl
from jax.experimental.pallas import tpu as pltpu
from jax.experimental.pallas import tpu_sc as plsc

F = 5
F8 = 8
HID = 64
DIN = 256
DOUT = 256
G = 100
NPG = 100
EPG = 1600
N = G * NPG
E = G * EPG

ROWS = 2000
NSLAB = 5
TCOL = 2048
EBLK = E // 128
NW = 32
LANES = 16
GPW = 4
NPAD = 112
SLAB = GPW * NPG
SEBLK = GPW * EPG // 128


def _tc1_body(x_ref, w1_ref, b1_ref, w2t_ref, b2t_ref, fv_ref):
    h = jnp.dot(x_ref[...], w1_ref[...], preferred_element_type=jnp.float32)
    h = h + b1_ref[...]
    h = h * jax.nn.sigmoid(h)
    fvT = lax.dot_general(w2t_ref[...], h, (((1,), (1,)), ((), ())),
                          preferred_element_type=jnp.float32)
    fvT = fvT + b2t_ref[...]
    fvT = jnp.concatenate(
        [fvT, jnp.zeros((F8, TCOL - ROWS), jnp.float32)], axis=1)
    fv_ref[...] = fvT.reshape(1, F8, TCOL)


def _tc1(x, W1, b1, W2pT, b2T):
    return pl.pallas_call(
        _tc1_body,
        grid=(N // ROWS,),
        in_specs=[
            pl.BlockSpec((ROWS, DIN), lambda i: (i, 0)),
            pl.BlockSpec((DIN, HID), lambda i: (0, 0)),
            pl.BlockSpec((1, HID), lambda i: (0, 0)),
            pl.BlockSpec((F8, HID), lambda i: (0, 0)),
            pl.BlockSpec((F8, 1), lambda i: (0, 0)),
        ],
        out_specs=pl.BlockSpec((1, F8, TCOL), lambda i: (i, 0, 0)),
        out_shape=jax.ShapeDtypeStruct((NSLAB, F8, TCOL), jnp.float32),
    )(x, W1, b1, W2pT, b2T)


def _tc2a_body(x_ref, wox_ref, bo_ref, acc_ref):
    acc_ref[...] = (
        jnp.dot(x_ref[...], wox_ref[...], preferred_element_type=jnp.float32)
        + bo_ref[...]
    )


def _tc2a(x, WoX, bo):
    return pl.pallas_call(
        _tc2a_body,
        grid=(N // ROWS,),
        in_specs=[
            pl.BlockSpec((ROWS, DIN), lambda i: (i, 0)),
            pl.BlockSpec((DIN, DOUT), lambda i: (0, 0)),
            pl.BlockSpec((1, DOUT), lambda i: (0, 0)),
        ],
        out_specs=pl.BlockSpec((ROWS, DOUT), lambda i: (i, 0)),
        out_shape=jax.ShapeDtypeStruct((N, DOUT), jnp.float32),
    )(x, WoX, bo)


def _sc_body(fv_hbm, ei_hbm, d0_hbm, gact_hbm,
             fv0, fv1, fv2, fv3, fv4, d00, d01, d02, d03, d04,
             ei_v, fe_v, gact_v, sem, *priv):
    fv_vf = [fv0, fv1, fv2, fv3, fv4]
    d0_vf = [d00, d01, d02, d03, d04]
    wid = lax.axis_index("s") * 2 + lax.axis_index("c")
    start = wid * GPW
    es = jnp.minimum(start, G - GPW)
    eg = es // GPW
    nlo = es * NPG
    lane = lax.broadcasted_iota(jnp.int32, (LANES,), 0)
    lane_np = lane * NPAD
    inf16 = jnp.full((LANES,), jnp.inf, jnp.float32)
    zero16 = jnp.zeros((LANES,), jnp.float32)
    ninf16 = jnp.full((LANES,), -jnp.inf, jnp.float32)

    segs = []
    for sg in range(2):
        m = nlo + sg * (SLAB // 2)
        slab = m // ROWS
        off = m - slab * ROWS
        segs.append((sg, slab, off))
    cin = []
    for f in range(F):
        for sg, slab, off in segs:
            cin.append(pltpu.make_async_copy(
                fv_hbm.at[slab, f, pl.ds(off, SLAB // 2)],
                fv_vf[f].at[pl.ds(sg * (SLAB // 2), SLAB // 2)], sem))
    cin.append(pltpu.make_async_copy(
        ei_hbm.at[pl.ds(eg * SEBLK, SEBLK)], ei_v, sem))
    for c in cin:
        c.start()

    @plsc.parallel_loop(0, LANES * NPAD // 16)
    def init_body(j):
        for p in priv:
            p[pl.ds(j * 16, 16)] = inf16

    for c in cin:
        c.wait()

    def do_graph(k, c):
        gbase = (es + k) * NPG
        rbase = k * NPG

        def pa(i, cc):
            fe0 = k * EPG + i * 16
            blk = fe0 // 128
            off = fe0 - blk * 128
            s = ei_v[blk, 0, pl.ds(off, 16)]
            t = ei_v[blk, 1, pl.ds(off, 16)]
            sl = s - gbase
            tl = t - gbase
            snd = rbase + sl
            tnd = rbase + tl
            ips = lane_np + sl
            ipt = lane_np + tl
            fes, css, cts = [], [], []
            for f in range(F):
                a = plsc.load_gather(fv_vf[f], [snd])
                b = plsc.load_gather(fv_vf[f], [tnd])
                fes.append(jnp.maximum(a, b))
            for f in range(F):
                css.append(plsc.load_gather(priv[f], [ips]))
                cts.append(plsc.load_gather(priv[F + f], [ipt]))
            for f in range(F):
                fe_v[pl.ds(f * EPG + i * 16, 16)] = fes[f]
            for f in range(F):
                plsc.store_scatter(priv[f], [ips], jnp.minimum(css[f], fes[f]))
                plsc.store_scatter(priv[F + f], [ipt], jnp.minimum(cts[f], fes[f]))
            return cc
        lax.fori_loop(0, EPG // 16, pa, 0)

        @plsc.parallel_loop(0, NPAD // 16)
        def rd(j):
            nidx = j * 16 + lane
            msk = nidx < NPG
            nd = rbase + nidx
            for f in range(F):
                vals = []
                for h in range(2):
                    pr = priv[h * F + f]
                    for l in range(LANES):
                        vals.append(pr[pl.ds(l * NPAD + j * 16, 16)])
                while len(vals) > 1:
                    vals = [jnp.minimum(vals[p], vals[p + 1])
                            for p in range(0, len(vals) - 1, 2)] + (
                                [vals[-1]] if len(vals) % 2 else [])
                m = vals[0]
                for h in range(2):
                    pr = priv[h * F + f]
                    for l in range(LANES):
                        pr[pl.ds(l * NPAD + j * 16, 16)] = inf16
                fv16 = plsc.load_gather(fv_vf[f], [nd], mask=msk)
                m = jnp.where(m == inf16, fv16, m)
                plsc.store_scatter(d0_vf[f], [nd], m, mask=msk)

        carry0 = ((zero16,) * F, (zero16,) * F, (ninf16,) * F)

        @plsc.parallel_loop(0, EPG // 16, carry=carry0)
        def pbout(i, carry):
            births, cnts, gmaxs = carry
            nb, nc, ng = list(births), list(cnts), list(gmaxs)
            fe0 = k * EPG + i * 16
            blk = fe0 // 128
            off = fe0 - blk * 128
            s = ei_v[blk, 0, pl.ds(off, 16)]
            t = ei_v[blk, 1, pl.ds(off, 16)]
            snd = rbase + s - gbase
            tnd = rbase + t - gbase
            for f in range(F):
                fe = fe_v[pl.ds(f * EPG + i * 16, 16)]
                dsv = plsc.load_gather(d0_vf[f], [snd])
                dtv = plsc.load_gather(d0_vf[f], [tnd])
                cyc = fe > jnp.maximum(dsv, dtv)
                nb[f] = nb[f] + jnp.where(cyc, fe, 0.0)
                nc[f] = nc[f] + jnp.where(cyc, 1.0, 0.0)
                ng[f] = jnp.maximum(ng[f], fe)
            return (tuple(nb), tuple(nc), tuple(ng))

        births, cnts, gmaxs = pbout

        v = zero16
        for f in range(F):
            bsum = jnp.sum(births[f])
            dsum = jnp.max(gmaxs[f]) * jnp.sum(cnts[f])
            v = jnp.where(lane == 2 * f, bsum, v)
            v = jnp.where(lane == 2 * f + 1, dsum, v)
        gact_v[pl.ds(k * 16, 16)] = v
        return c

    lax.fori_loop(0, GPW, do_graph, 0)

    cout = []
    for f in range(F):
        for sg, slab, off in segs:
            cout.append(pltpu.make_async_copy(
                d0_vf[f].at[pl.ds(sg * (SLAB // 2), SLAB // 2)],
                d0_hbm.at[slab, f, pl.ds(off, SLAB // 2)], sem))
    cout.append(pltpu.make_async_copy(
        gact_v, gact_hbm.at[pl.ds(pl.multiple_of(es * 16, 8), GPW * 16)], sem))
    for c in cout:
        c.start()
    for c in cout:
        c.wait()


def _sc_edges(fv_p, ei_p):
    fn = pl.kernel(
        _sc_body,
        out_type=[
            jax.ShapeDtypeStruct((NSLAB, F8, TCOL), jnp.float32),
            jax.ShapeDtypeStruct((G * 16,), jnp.float32),
        ],
        mesh=plsc.VectorSubcoreMesh(core_axis_name="c", subcore_axis_name="s"),
        compiler_params=pltpu.CompilerParams(
            needs_layout_passes=False, use_tc_tiling_on_sc=False),
        scratch_types=(
            [pltpu.VMEM((SLAB,), jnp.float32) for _ in range(F)]
            + [pltpu.VMEM((SLAB,), jnp.float32) for _ in range(F)]
            + [
                pltpu.VMEM((SEBLK, 2, 128), jnp.int32),
                pltpu.VMEM((F * EPG,), jnp.float32),
                pltpu.VMEM((GPW * 16,), jnp.float32),
                pltpu.SemaphoreType.DMA,
            ]
            + [pltpu.VMEM((LANES * NPAD,), jnp.float32) for _ in range(2 * F)]
        ),
    )
    return fn(fv_p, ei_p)


def _tc2b_body(acc_ref, fv_ref, d0_ref, wga_ref, wgb_ref,
               mua_ref, nia_ref, mub_ref, nib_ref, out_ref):
    fvT = fv_ref[...].reshape(F8, TCOL)
    d0T = d0_ref[...].reshape(F8, TCOL)
    frow = lax.broadcasted_iota(jnp.int32, (F8, TCOL), 0)
    d0T = jnp.where(frow < F, d0T, 0.0)
    gaT = jnp.exp(nia_ref[...] * (fvT - mua_ref[...]) ** 2)
    gbT = jnp.exp(nib_ref[...] * (d0T - mub_ref[...]) ** 2)
    ge = lax.dot_general(gaT, wga_ref[...], (((0,), (0,)), ((), ())),
                         preferred_element_type=jnp.float32)
    go = lax.dot_general(gbT, wgb_ref[...], (((0,), (0,)), ((), ())),
                         preferred_element_type=jnp.float32)
    a = acc_ref[...] + ge[:ROWS] + go[:ROWS]
    out_ref[...] = a * jax.nn.sigmoid(a)


def _tc2b(acc, fv_p, d0_p, WGA, WGB, muA, niA, muB, niB):
    return pl.pallas_call(
        _tc2b_body,
        grid=(N // ROWS,),
        in_specs=[
            pl.BlockSpec((ROWS, DOUT), lambda i: (i, 0)),
            pl.BlockSpec((1, F8, TCOL), lambda i: (i, 0, 0)),
            pl.BlockSpec((1, F8, TCOL), lambda i: (i, 0, 0)),
            pl.BlockSpec((F8, DOUT), lambda i: (0, 0)),
            pl.BlockSpec((F8, DOUT), lambda i: (0, 0)),
            pl.BlockSpec((F8, 1), lambda i: (0, 0)),
            pl.BlockSpec((F8, 1), lambda i: (0, 0)),
            pl.BlockSpec((F8, 1), lambda i: (0, 0)),
            pl.BlockSpec((F8, 1), lambda i: (0, 0)),
        ],
        out_specs=pl.BlockSpec((ROWS, DOUT), lambda i: (i, 0)),
        out_shape=jax.ShapeDtypeStruct((N, DOUT), jnp.float32),
    )(acc, fv_p, d0_p, WGA, WGB, muA, niA, muB, niB)


def kernel(x, edge_index, x_slices, edge_slices, W1, b1, W2, b2, mu, sigma, Wo, bo):
    f32 = jnp.float32
    W2pT = jnp.zeros((F8, HID), f32).at[:F].set(W2.T)
    b2T = jnp.zeros((F8, 1), f32).at[:F, 0].set(b2)
    fv_p = _tc1(x, W1, b1.reshape(1, HID), W2pT, b2T)
    acc = _tc2a(x, Wo[:DIN], bo.reshape(1, DOUT))

    ei_p = edge_index.reshape(2, EBLK, 128).transpose(1, 0, 2)

    d0_p, gact_flat = _sc_edges(fv_p, ei_p)

    muA = jnp.zeros((F8, 1), f32).at[:F, 0].set(mu[0::2])
    muB = jnp.zeros((F8, 1), f32).at[:F, 0].set(mu[1::2])
    niA = jnp.zeros((F8, 1), f32).at[:F, 0].set(-0.5 / (sigma[0::2] ** 2))
    niB = jnp.zeros((F8, 1), f32).at[:F, 0].set(-0.5 / (sigma[1::2] ** 2))
    WGA = jnp.zeros((F8, DOUT), f32).at[:F].set(Wo[DIN::2])
    WGB = jnp.zeros((F8, DOUT), f32).at[:F].set(Wo[DIN + 1::2])

    out = _tc2b(acc, fv_p, d0_p, WGA, WGB, muA, niA, muB, niB)
    gact = gact_flat.reshape(G, 16)[:, : 2 * F]
    return out, gact


# speedup vs baseline: 2.5955x; 1.0135x over previous
import jax
import jax.numpy as jnp
from jax import lax
from jax.experimental import pallas as pl
from jax.experimental.pallas import tpu as pltpu
from jax.experimental.pallas import tpu_sc as plsc

F = 5
F8 = 8
HID = 64
DIN = 256
DOUT = 256
G = 100
NPG = 100
EPG = 1600
N = G * NPG
E = G * EPG

ROWS = 2000
NSLAB = 5
TCOL = 2048
EBLK = E // 128
NW = 32
LANES = 16
GPW = 4
NPAD = 112
SLAB = GPW * NPG
SEBLK = GPW * EPG // 128


def _tc1_body(x_ref, w1_ref, b1_ref, w2t_ref, b2t_ref, fv_ref):
    h = jnp.dot(x_ref[...], w1_ref[...], preferred_element_type=jnp.float32)
    h = h + b1_ref[...]
    h = h * jax.nn.sigmoid(h)
    fvT = lax.dot_general(w2t_ref[...], h, (((1,), (1,)), ((), ())),
                          preferred_element_type=jnp.float32)
    fvT = fvT + b2t_ref[...]
    fvT = jnp.concatenate(
        [fvT, jnp.zeros((F8, TCOL - ROWS), jnp.float32)], axis=1)
    fv_ref[...] = fvT.reshape(1, F8, TCOL)


def _tc1(x, W1, b1, W2pT, b2T):
    return pl.pallas_call(
        _tc1_body,
        grid=(N // ROWS,),
        in_specs=[
            pl.BlockSpec((ROWS, DIN), lambda i: (i, 0)),
            pl.BlockSpec((DIN, HID), lambda i: (0, 0)),
            pl.BlockSpec((1, HID), lambda i: (0, 0)),
            pl.BlockSpec((F8, HID), lambda i: (0, 0)),
            pl.BlockSpec((F8, 1), lambda i: (0, 0)),
        ],
        out_specs=pl.BlockSpec((1, F8, TCOL), lambda i: (i, 0, 0)),
        out_shape=jax.ShapeDtypeStruct((NSLAB, F8, TCOL), jnp.float32),
    )(x, W1, b1, W2pT, b2T)


def _tc2a_body(x_ref, wox_ref, bo_ref, acc_ref):
    acc_ref[...] = (
        jnp.dot(x_ref[...], wox_ref[...], preferred_element_type=jnp.float32)
        + bo_ref[...]
    )


def _tc2a(x, WoX, bo):
    return pl.pallas_call(
        _tc2a_body,
        grid=(N // ROWS,),
        in_specs=[
            pl.BlockSpec((ROWS, DIN), lambda i: (i, 0)),
            pl.BlockSpec((DIN, DOUT), lambda i: (0, 0)),
            pl.BlockSpec((1, DOUT), lambda i: (0, 0)),
        ],
        out_specs=pl.BlockSpec((ROWS, DOUT), lambda i: (i, 0)),
        out_shape=jax.ShapeDtypeStruct((N, DOUT), jnp.float32),
    )(x, WoX, bo)


def _sc_body(fv_hbm, ei_hbm, d0_hbm, gact_hbm,
             fv0, fv1, fv2, fv3, fv4, d00, d01, d02, d03, d04,
             ei_v, fe_v, gact_v, sem, *priv):
    fv_vf = [fv0, fv1, fv2, fv3, fv4]
    d0_vf = [d00, d01, d02, d03, d04]
    wid = lax.axis_index("s") * 2 + lax.axis_index("c")
    start = wid * GPW
    es = jnp.minimum(start, G - GPW)
    eg = es // GPW
    nlo = es * NPG
    lane = lax.broadcasted_iota(jnp.int32, (LANES,), 0)
    lane_np = lane * NPAD
    inf16 = jnp.full((LANES,), jnp.inf, jnp.float32)
    zero16 = jnp.zeros((LANES,), jnp.float32)
    ninf16 = jnp.full((LANES,), -jnp.inf, jnp.float32)

    segs = []
    for sg in range(2):
        m = nlo + sg * (SLAB // 2)
        slab = m // ROWS
        off = m - slab * ROWS
        segs.append((sg, slab, off))
    cin = []
    for f in range(F):
        for sg, slab, off in segs:
            cin.append(pltpu.make_async_copy(
                fv_hbm.at[slab, f, pl.ds(off, SLAB // 2)],
                fv_vf[f].at[pl.ds(sg * (SLAB // 2), SLAB // 2)], sem))
    cin.append(pltpu.make_async_copy(
        ei_hbm.at[pl.ds(eg * SEBLK, SEBLK)], ei_v, sem))
    for c in cin:
        c.start()

    @plsc.parallel_loop(0, LANES * NPAD // 16)
    def init_body(j):
        for p in priv:
            p[pl.ds(j * 16, 16)] = inf16

    for c in cin:
        c.wait()

    def do_graph(k, c):
        gbase = (es + k) * NPG
        rbase = k * NPG

        def pa(i, cc):
            fe0 = k * EPG + i * 16
            blk = fe0 // 128
            off = fe0 - blk * 128
            s = ei_v[blk, 0, pl.ds(off, 16)]
            t = ei_v[blk, 1, pl.ds(off, 16)]
            sl = s - gbase
            tl = t - gbase
            snd = rbase + sl
            tnd = rbase + tl
            ips = lane_np + sl
            ipt = lane_np + tl
            fes, css, cts = [], [], []
            for f in range(F):
                a = plsc.load_gather(fv_vf[f], [snd])
                b = plsc.load_gather(fv_vf[f], [tnd])
                fes.append(jnp.maximum(a, b))
            for f in range(F):
                css.append(plsc.load_gather(priv[f], [ips]))
                cts.append(plsc.load_gather(priv[F + f], [ipt]))
            for f in range(F):
                fe_v[pl.ds(f * EPG + i * 16, 16)] = fes[f]
            for f in range(F):
                plsc.store_scatter(priv[f], [ips], jnp.minimum(css[f], fes[f]))
                plsc.store_scatter(priv[F + f], [ipt], jnp.minimum(cts[f], fes[f]))
            return cc
        lax.fori_loop(0, EPG // 16, pa, 0, unroll=2)

        @plsc.parallel_loop(0, NPAD // 16)
        def rd(j):
            nidx = j * 16 + lane
            msk = nidx < NPG
            nd = rbase + nidx
            for f in range(F):
                accs = [None] * 4
                for h in range(2):
                    pr = priv[h * F + f]
                    for l in range(LANES):
                        v = pr[pl.ds(l * NPAD + j * 16, 16)]
                        q = l & 3
                        accs[q] = v if accs[q] is None and h == 0 else (
                            jnp.minimum(accs[q], v))
                m = jnp.minimum(jnp.minimum(accs[0], accs[1]),
                                jnp.minimum(accs[2], accs[3]))
                for h in range(2):
                    pr = priv[h * F + f]
                    for l in range(LANES):
                        pr[pl.ds(l * NPAD + j * 16, 16)] = inf16
                fv16 = plsc.load_gather(fv_vf[f], [nd], mask=msk)
                m = jnp.where(m == inf16, fv16, m)
                plsc.store_scatter(d0_vf[f], [nd], m, mask=msk)

        carry0 = ((zero16,) * F, (zero16,) * F, (ninf16,) * F)

        @plsc.parallel_loop(0, EPG // 16, carry=carry0)
        def pbout(i, carry):
            births, cnts, gmaxs = carry
            nb, nc, ng = list(births), list(cnts), list(gmaxs)
            fe0 = k * EPG + i * 16
            blk = fe0 // 128
            off = fe0 - blk * 128
            s = ei_v[blk, 0, pl.ds(off, 16)]
            t = ei_v[blk, 1, pl.ds(off, 16)]
            snd = rbase + s - gbase
            tnd = rbase + t - gbase
            for f in range(F):
                fe = fe_v[pl.ds(f * EPG + i * 16, 16)]
                dsv = plsc.load_gather(d0_vf[f], [snd])
                dtv = plsc.load_gather(d0_vf[f], [tnd])
                cyc = fe > jnp.maximum(dsv, dtv)
                nb[f] = nb[f] + jnp.where(cyc, fe, 0.0)
                nc[f] = nc[f] + jnp.where(cyc, 1.0, 0.0)
                ng[f] = jnp.maximum(ng[f], fe)
            return (tuple(nb), tuple(nc), tuple(ng))

        births, cnts, gmaxs = pbout

        v = zero16
        for f in range(F):
            bsum = jnp.sum(births[f])
            dsum = jnp.max(gmaxs[f]) * jnp.sum(cnts[f])
            v = jnp.where(lane == 2 * f, bsum, v)
            v = jnp.where(lane == 2 * f + 1, dsum, v)
        gact_v[pl.ds(k * 16, 16)] = v
        return c

    lax.fori_loop(0, GPW, do_graph, 0)

    cout = []
    for f in range(F):
        for sg, slab, off in segs:
            cout.append(pltpu.make_async_copy(
                d0_vf[f].at[pl.ds(sg * (SLAB // 2), SLAB // 2)],
                d0_hbm.at[slab, f, pl.ds(off, SLAB // 2)], sem))
    cout.append(pltpu.make_async_copy(
        gact_v, gact_hbm.at[pl.ds(pl.multiple_of(es * 16, 8), GPW * 16)], sem))
    for c in cout:
        c.start()
    for c in cout:
        c.wait()


def _sc_edges(fv_p, ei_p):
    fn = pl.kernel(
        _sc_body,
        out_type=[
            jax.ShapeDtypeStruct((NSLAB, F8, TCOL), jnp.float32),
            jax.ShapeDtypeStruct((G * 16,), jnp.float32),
        ],
        mesh=plsc.VectorSubcoreMesh(core_axis_name="c", subcore_axis_name="s"),
        compiler_params=pltpu.CompilerParams(
            needs_layout_passes=False, use_tc_tiling_on_sc=False),
        scratch_types=(
            [pltpu.VMEM((SLAB,), jnp.float32) for _ in range(F)]
            + [pltpu.VMEM((SLAB,), jnp.float32) for _ in range(F)]
            + [
                pltpu.VMEM((SEBLK, 2, 128), jnp.int32),
                pltpu.VMEM((F * EPG,), jnp.float32),
                pltpu.VMEM((GPW * 16,), jnp.float32),
                pltpu.SemaphoreType.DMA,
            ]
            + [pltpu.VMEM((LANES * NPAD,), jnp.float32) for _ in range(2 * F)]
        ),
    )
    return fn(fv_p, ei_p)


def _tc2b_body(acc_ref, fv_ref, d0_ref, wga_ref, wgb_ref,
               mua_ref, nia_ref, mub_ref, nib_ref, out_ref):
    fvT = fv_ref[...].reshape(F8, TCOL)
    d0T = d0_ref[...].reshape(F8, TCOL)
    frow = lax.broadcasted_iota(jnp.int32, (F8, TCOL), 0)
    d0T = jnp.where(frow < F, d0T, 0.0)
    gaT = jnp.exp(nia_ref[...] * (fvT - mua_ref[...]) ** 2)
    gbT = jnp.exp(nib_ref[...] * (d0T - mub_ref[...]) ** 2)
    ge = lax.dot_general(gaT, wga_ref[...], (((0,), (0,)), ((), ())),
                         preferred_element_type=jnp.float32)
    go = lax.dot_general(gbT, wgb_ref[...], (((0,), (0,)), ((), ())),
                         preferred_element_type=jnp.float32)
    a = acc_ref[...] + ge[:ROWS] + go[:ROWS]
    out_ref[...] = a * jax.nn.sigmoid(a)


def _tc2b(acc, fv_p, d0_p, WGA, WGB, muA, niA, muB, niB):
    return pl.pallas_call(
        _tc2b_body,
        grid=(N // ROWS,),
        in_specs=[
            pl.BlockSpec((ROWS, DOUT), lambda i: (i, 0)),
            pl.BlockSpec((1, F8, TCOL), lambda i: (i, 0, 0)),
            pl.BlockSpec((1, F8, TCOL), lambda i: (i, 0, 0)),
            pl.BlockSpec((F8, DOUT), lambda i: (0, 0)),
            pl.BlockSpec((F8, DOUT), lambda i: (0, 0)),
            pl.BlockSpec((F8, 1), lambda i: (0, 0)),
            pl.BlockSpec((F8, 1), lambda i: (0, 0)),
            pl.BlockSpec((F8, 1), lambda i: (0, 0)),
            pl.BlockSpec((F8, 1), lambda i: (0, 0)),
        ],
        out_specs=pl.BlockSpec((ROWS, DOUT), lambda i: (i, 0)),
        out_shape=jax.ShapeDtypeStruct((N, DOUT), jnp.float32),
    )(acc, fv_p, d0_p, WGA, WGB, muA, niA, muB, niB)


def kernel(x, edge_index, x_slices, edge_slices, W1, b1, W2, b2, mu, sigma, Wo, bo):
    f32 = jnp.float32
    W2pT = jnp.zeros((F8, HID), f32).at[:F].set(W2.T)
    b2T = jnp.zeros((F8, 1), f32).at[:F, 0].set(b2)
    fv_p = _tc1(x, W1, b1.reshape(1, HID), W2pT, b2T)
    acc = _tc2a(x, Wo[:DIN], bo.reshape(1, DOUT))

    ei_p = edge_index.reshape(2, EBLK, 128).transpose(1, 0, 2)

    d0_p, gact_flat = _sc_edges(fv_p, ei_p)

    muA = jnp.zeros((F8, 1), f32).at[:F, 0].set(mu[0::2])
    muB = jnp.zeros((F8, 1), f32).at[:F, 0].set(mu[1::2])
    niA = jnp.zeros((F8, 1), f32).at[:F, 0].set(-0.5 / (sigma[0::2] ** 2))
    niB = jnp.zeros((F8, 1), f32).at[:F, 0].set(-0.5 / (sigma[1::2] ** 2))
    WGA = jnp.zeros((F8, DOUT), f32).at[:F].set(Wo[DIN::2])
    WGB = jnp.zeros((F8, DOUT), f32).at[:F].set(Wo[DIN + 1::2])

    out = _tc2b(acc, fv_p, d0_p, WGA, WGB, muA, niA, muB, niB)
    gact = gact_flat.reshape(G, 16)[:, : 2 * F]
    return out, gact
